# Initial kernel scaffold; baseline (speedup 1.0000x reference)
import jax
import jax.numpy as jnp
from jax.experimental import pallas as pl


def kernel(x, pos, edge_index, W1a, b1a, W1b, b1b, W2a, b2a, W2b, b2b):
    raise NotImplementedError("write your pallas kernel here")


# speedup vs baseline: 2.1471x; 2.1471x over previous
import functools

import jax
import jax.numpy as jnp
from jax import lax
from jax.experimental import pallas as pl
from jax.experimental.pallas import tpu as pltpu
from jax.experimental.pallas import tpu_sc as plsc

N = 10000
E = 320000
F = 128
H = 64
NC, NS = 2, 16
NW = NC * NS
CH = 128
EPAD = 327680
PERW = EPAD // NW
NCH = PERW // CH
NPAD = 10240
TW = 144
BE = 4096


def _tables_body(x_ref, pos_ref, wr_ref, wc_ref, tr_ref, tc_ref):
    x = x_ref[...]
    p3 = pos_ref[...]
    pad = jnp.zeros((x.shape[0], TW - F - 3), jnp.float32)
    tr_ref[...] = jnp.concatenate(
        [jnp.dot(x, wr_ref[...], preferred_element_type=jnp.float32), p3, pad],
        axis=1)
    tc_ref[...] = jnp.concatenate(
        [jnp.dot(x, wc_ref[...], preferred_element_type=jnp.float32), p3, pad],
        axis=1)


def _node_tables(x, pos, Wr, Wc):
    blk = 1000
    return pl.pallas_call(
        _tables_body,
        grid=(N // blk,),
        in_specs=[pl.BlockSpec((blk, F), lambda i: (i, 0)),
                  pl.BlockSpec((blk, 3), lambda i: (i, 0)),
                  pl.BlockSpec((F, F), lambda i: (0, 0)),
                  pl.BlockSpec((F, F), lambda i: (0, 0))],
        out_specs=[pl.BlockSpec((blk, TW), lambda i: (i, 0)),
                   pl.BlockSpec((blk, TW), lambda i: (i, 0))],
        out_shape=[jax.ShapeDtypeStruct((N, TW), jnp.float32)] * 2,
    )(x, pos, Wr, Wc)


_SC_PARAMS = pltpu.CompilerParams(use_tc_tiling_on_sc=False)


def _sc_gather(tr, tc_, row, col):
    mesh = plsc.VectorSubcoreMesh(core_axis_name="c", subcore_axis_name="s")

    @functools.partial(
        pl.kernel, mesh=mesh,
        compiler_params=_SC_PARAMS,
        out_type=[jax.ShapeDtypeStruct((EPAD, TW), jnp.float32)] * 2,
        scratch_types=[pltpu.VMEM((CH,), jnp.int32),
                       pltpu.VMEM((CH,), jnp.int32),
                       pltpu.VMEM((CH, TW), jnp.float32),
                       pltpu.VMEM((CH, TW), jnp.float32),
                       pltpu.SemaphoreType.DMA,
                       pltpu.SemaphoreType.DMA],
    )
    def k(tr_hbm, tc_hbm, row_hbm, col_hbm, gr_hbm, gc_hbm,
          idxr, idxc, bufr, bufc, sem1, sem2):
        wid = lax.axis_index("s") * NC + lax.axis_index("c")
        base = wid * PERW

        @pl.loop(0, NCH)
        def _(i):
            off = base + i * CH
            pltpu.sync_copy(row_hbm.at[pl.ds(off, CH)], idxr)
            pltpu.sync_copy(col_hbm.at[pl.ds(off, CH)], idxc)
            c1 = pltpu.async_copy(tr_hbm.at[idxr], bufr, sem1)
            c2 = pltpu.async_copy(tc_hbm.at[idxc], bufc, sem2)
            c1.wait()
            c2.wait()
            pltpu.sync_copy(bufr, gr_hbm.at[pl.ds(off, CH)])
            pltpu.sync_copy(bufc, gc_hbm.at[pl.ds(off, CH)])

    return k(tr, tc_, row, col)


def _edge_body(gr_ref, gc_ref, pp_ref, v_ref):
    gr = gr_ref[...]
    gc = gc_ref[...]
    pr = pp_ref[...]
    a1 = gr[:, 0:H]
    a2 = gr[:, H:2 * H]
    b1 = gc[:, 0:H]
    b2 = gc[:, H:2 * H]
    d3 = gr[:, 2 * H:2 * H + 16] - gc[:, 2 * H:2 * H + 16]
    dist = jnp.sqrt(jnp.sum(d3 * d3, axis=1, keepdims=True))
    w1b = pr[0:1, 0:H]
    w2b = pr[1:2, 0:H]
    w1d = pr[2:3, 0:H]
    w2d = pr[3:4, 0:H]
    c1a = pr[4:5, 0:H]
    c2a = pr[5:6, 0:H]
    b1b = pr[6:7, 0:1]
    b2b = pr[6:7, 1:2]
    h1 = a1 + b1 + dist * w1d + c1a
    h2 = a2 + b2 + dist * w2d + c2a
    s1 = h1 * jax.nn.sigmoid(h1)
    s2 = h2 * jax.nn.sigmoid(h2)
    mes1 = jnp.sum(s1 * w1b, axis=1, keepdims=True) + b1b
    mes2 = jnp.sum(s2 * w2b, axis=1, keepdims=True) + b2b
    rmax = 4.5
    t = jnp.clip(dist, 0.0, rmax) / rmax
    t2 = t * t
    t4 = t2 * t2
    t5 = t4 * t
    t6 = t5 * t
    t7 = t6 * t
    coe = 1.0 - 21.0 * t5 + 35.0 * t6 - 15.0 * t7
    inv = coe / (dist + 1e-6)
    v1 = d3 * (inv * mes1)
    v2 = d3 * (inv * mes2)
    v_ref[...] = jnp.concatenate([v1[:, 0:8], v2[:, 0:8]], axis=1)


def _edge_math(gr, gc, params):
    return pl.pallas_call(
        _edge_body,
        grid=(EPAD // BE,),
        in_specs=[pl.BlockSpec((BE, TW), lambda i: (i, 0)),
                  pl.BlockSpec((BE, TW), lambda i: (i, 0)),
                  pl.BlockSpec((8, 128), lambda i: (0, 0))],
        out_specs=pl.BlockSpec((BE, 16), lambda i: (i, 0)),
        out_shape=jax.ShapeDtypeStruct((EPAD, 16), jnp.float32),
    )(gr, gc, params)


def _sc_scatter(v, col):
    mesh = plsc.VectorSubcoreMesh(core_axis_name="c", subcore_axis_name="s")
    ZR = NPAD // NS

    @functools.partial(
        pl.kernel, mesh=mesh,
        compiler_params=_SC_PARAMS,
        out_type=jax.ShapeDtypeStruct((NC, NPAD, 16), jnp.float32),
        scratch_types=[pltpu.VMEM((CH, 16), jnp.float32),
                       pltpu.VMEM((CH,), jnp.int32),
                       pltpu.VMEM_SHARED((NPAD, 16), jnp.float32)],
    )
    def k(v_hbm, col_hbm, out_hbm, vbuf, idxbuf, acc):
        cid = lax.axis_index("c")
        sid = lax.axis_index("s")
        wid = sid * NC + cid

        @pl.loop(0, CH)
        def _(i):
            vbuf[i, :] = jnp.zeros((16,), jnp.float32)

        @pl.loop(0, ZR // CH)
        def _(j):
            pltpu.sync_copy(vbuf, acc.at[pl.ds(sid * ZR + j * CH, CH)])

        plsc.subcore_barrier()

        @pl.loop(0, NCH)
        def _(i):
            off = wid * PERW + i * CH
            pltpu.sync_copy(v_hbm.at[pl.ds(off, CH)], vbuf)
            pltpu.sync_copy(col_hbm.at[pl.ds(off, CH)], idxbuf)
            pltpu.sync_copy(vbuf, acc.at[idxbuf], add=True)

        plsc.subcore_barrier()
        pltpu.sync_copy(acc.at[pl.ds(sid * ZR, ZR)],
                        out_hbm.at[cid, pl.ds(sid * ZR, ZR)])

    return k(v, col)


def _fin_body(p_ref, o_ref):
    p = p_ref[0] + p_ref[1]
    eps = 1e-6
    v1 = p[:, 0:3]
    v2 = p[:, 8:11]
    v1n = jnp.sqrt(jnp.sum(v1 * v1, axis=1, keepdims=True))
    one = jnp.ones_like(v1n)
    zero = jnp.zeros_like(v1n)
    default = jnp.concatenate([one, zero, zero], axis=1)
    n1 = jnp.where(v1n > eps, v1 / (v1n + eps), default)
    n2p = v2 - jnp.sum(n1 * v2, axis=1, keepdims=True) * n1
    n2n = jnp.sqrt(jnp.sum(n2p * n2p, axis=1, keepdims=True))
    fb = jnp.concatenate([-n1[:, 1:2], n1[:, 0:1], zero], axis=1)
    fb = fb - jnp.sum(n1 * fb, axis=1, keepdims=True) * n1
    fbn = jnp.sqrt(jnp.sum(fb * fb, axis=1, keepdims=True))
    fb = fb / (fbn + eps)
    n2 = jnp.where(n2n > eps, n2p / (n2n + eps), fb)
    c0 = n1[:, 1:2] * n2[:, 2:3] - n1[:, 2:3] * n2[:, 1:2]
    c1 = n1[:, 2:3] * n2[:, 0:1] - n1[:, 0:1] * n2[:, 2:3]
    c2 = n1[:, 0:1] * n2[:, 1:2] - n1[:, 1:2] * n2[:, 0:1]
    n3 = jnp.concatenate([c0, c1, c2], axis=1)
    n3n = jnp.sqrt(jnp.sum(n3 * n3, axis=1, keepdims=True))
    n3 = n3 / (n3n + eps)
    pad7 = jnp.zeros((p.shape[0], 7), jnp.float32)
    o_ref[...] = jnp.concatenate(
        [n1[:, 0:1], n2[:, 0:1], n3[:, 0:1],
         n1[:, 1:2], n2[:, 1:2], n3[:, 1:2],
         n1[:, 2:3], n2[:, 2:3], n3[:, 2:3], pad7], axis=1)


def _finalize(p):
    blk = 1024
    return pl.pallas_call(
        _fin_body,
        grid=(NPAD // blk,),
        in_specs=[pl.BlockSpec((NC, blk, 16), lambda i: (0, i, 0))],
        out_specs=pl.BlockSpec((blk, 16), lambda i: (i, 0)),
        out_shape=jax.ShapeDtypeStruct((NPAD, 16), jnp.float32),
    )(p)


def kernel(x, pos, edge_index, W1a, b1a, W1b, b1b, W2a, b2a, W2b, b2b):
    Wr = jnp.concatenate([W1a[:F], W2a[:F]], axis=1)
    Wc = jnp.concatenate([W1a[F:2 * F], W2a[F:2 * F]], axis=1)
    params = jnp.zeros((8, 128), jnp.float32)
    params = params.at[0, :H].set(W1b[:, 0])
    params = params.at[1, :H].set(W2b[:, 0])
    params = params.at[2, :H].set(W1a[2 * F])
    params = params.at[3, :H].set(W2a[2 * F])
    params = params.at[4, :H].set(b1a)
    params = params.at[5, :H].set(b2a)
    params = params.at[6, 0].set(b1b[0])
    params = params.at[6, 1].set(b2b[0])
    padlen = EPAD - E
    row = jnp.concatenate([edge_index[0], jnp.zeros((padlen,), jnp.int32)])
    col = jnp.concatenate([edge_index[1], jnp.zeros((padlen,), jnp.int32)])
    tr, tc_ = _node_tables(x, pos, Wr, Wc)
    gr, gc = _sc_gather(tr, tc_, row, col)
    v = _edge_math(gr, gc, params)
    p = _sc_scatter(v, col)
    o = _finalize(p)
    return o[:N, :9].reshape(N, 3, 3)


# speedup vs baseline: 2.3575x; 1.0980x over previous
import functools

import jax
import jax.numpy as jnp
from jax import lax
from jax.experimental import pallas as pl
from jax.experimental.pallas import tpu as pltpu
from jax.experimental.pallas import tpu_sc as plsc

N = 10000
E = 320000
F = 128
H = 64
NC, NS = 2, 16
NW = NC * NS
CH = 128
EPAD = 327680
PERW = EPAD // NW
NCH = PERW // CH
NPAD = 10240
TW = 144
BE = 4096


def _tables_body(x_ref, pos_ref, wr_ref, wc_ref, tr_ref, tc_ref):
    x = x_ref[...]
    p3 = pos_ref[...]
    pad = jnp.zeros((x.shape[0], TW - F - 3), jnp.float32)
    tr_ref[...] = jnp.concatenate(
        [jnp.dot(x, wr_ref[...], preferred_element_type=jnp.float32), p3, pad],
        axis=1)
    tc_ref[...] = jnp.concatenate(
        [jnp.dot(x, wc_ref[...], preferred_element_type=jnp.float32), p3, pad],
        axis=1)


def _node_tables(x, pos, Wr, Wc):
    blk = 1000
    return pl.pallas_call(
        _tables_body,
        grid=(N // blk,),
        in_specs=[pl.BlockSpec((blk, F), lambda i: (i, 0)),
                  pl.BlockSpec((blk, 3), lambda i: (i, 0)),
                  pl.BlockSpec((F, F), lambda i: (0, 0)),
                  pl.BlockSpec((F, F), lambda i: (0, 0))],
        out_specs=[pl.BlockSpec((blk, TW), lambda i: (i, 0)),
                   pl.BlockSpec((blk, TW), lambda i: (i, 0))],
        out_shape=[jax.ShapeDtypeStruct((N, TW), jnp.float32)] * 2,
    )(x, pos, Wr, Wc)


_SC_PARAMS = pltpu.CompilerParams(use_tc_tiling_on_sc=False)


def _sc_gather(tr, tc_, row3, col3):
    mesh = plsc.VectorSubcoreMesh(core_axis_name="c", subcore_axis_name="s")

    @functools.partial(
        pl.kernel, mesh=mesh,
        compiler_params=_SC_PARAMS,
        out_type=[jax.ShapeDtypeStruct((EPAD, TW), jnp.float32)] * 2,
        scratch_types=[pltpu.VMEM((NCH, CH), jnp.int32),
                       pltpu.VMEM((NCH, CH), jnp.int32),
                       pltpu.VMEM((CH, TW), jnp.float32),
                       pltpu.VMEM((CH, TW), jnp.float32),
                       pltpu.VMEM((CH, TW), jnp.float32),
                       pltpu.VMEM((CH, TW), jnp.float32),
                       pltpu.SemaphoreType.DMA,
                       pltpu.SemaphoreType.DMA,
                       pltpu.SemaphoreType.DMA,
                       pltpu.SemaphoreType.DMA,
                       pltpu.SemaphoreType.DMA,
                       pltpu.SemaphoreType.DMA,
                       pltpu.SemaphoreType.DMA,
                       pltpu.SemaphoreType.DMA,
                       pltpu.SemaphoreType.DMA],
    )
    def k(tr_hbm, tc_hbm, row_hbm, col_hbm, gr_hbm, gc_hbm,
          idxr, idxc, br0, bc0, br1, bc1,
          gsr0, gsc0, gsr1, gsc1, wsr0, wsc0, wsr1, wsc1, isem):
        wid = lax.axis_index("s") * NC + lax.axis_index("c")
        base = wid * PERW
        pltpu.async_copy(row_hbm.at[wid], idxr, isem).wait()
        pltpu.async_copy(col_hbm.at[wid], idxc, isem).wait()
        bufs = ((br0, bc0, gsr0, gsc0, wsr0, wsc0),
                (br1, bc1, gsr1, gsc1, wsr1, wsc1))

        def start_gather(j, p):
            br, bc, gsr, gsc, _, _ = bufs[p]
            pltpu.make_async_copy(tr_hbm.at[idxr.at[j]], br, gsr).start()
            pltpu.make_async_copy(tc_hbm.at[idxc.at[j]], bc, gsc).start()

        def wait_gather(j, p):
            br, bc, gsr, gsc, _, _ = bufs[p]
            pltpu.make_async_copy(tr_hbm.at[idxr.at[j]], br, gsr).wait()
            pltpu.make_async_copy(tc_hbm.at[idxc.at[j]], bc, gsc).wait()

        def start_write(j, p):
            br, bc, _, _, wsr, wsc = bufs[p]
            off = base + j * CH
            pltpu.make_async_copy(br, gr_hbm.at[pl.ds(off, CH)], wsr).start()
            pltpu.make_async_copy(bc, gc_hbm.at[pl.ds(off, CH)], wsc).start()

        def wait_write(j, p):
            br, bc, _, _, wsr, wsc = bufs[p]
            off = base + j * CH
            pltpu.make_async_copy(br, gr_hbm.at[pl.ds(off, CH)], wsr).wait()
            pltpu.make_async_copy(bc, gc_hbm.at[pl.ds(off, CH)], wsc).wait()

        start_gather(0, 0)
        start_gather(1, 1)
        wait_gather(0, 0)
        start_write(0, 0)
        wait_gather(1, 1)
        start_write(1, 1)

        @pl.loop(1, NCH // 2)
        def _(i):
            j0 = 2 * i
            for p in range(2):
                j = j0 + p
                wait_write(j, p)
                start_gather(j, p)
                wait_gather(j, p)
                start_write(j, p)

        wait_write(NCH - 2, 0)
        wait_write(NCH - 1, 1)

    return k(tr, tc_, row3, col3)


def _edge_body(gr_ref, gc_ref, pp_ref, v_ref):
    gr = gr_ref[...]
    gc = gc_ref[...]
    pr = pp_ref[...]
    a1 = gr[:, 0:H]
    a2 = gr[:, H:2 * H]
    b1 = gc[:, 0:H]
    b2 = gc[:, H:2 * H]
    d3 = gr[:, 2 * H:2 * H + 16] - gc[:, 2 * H:2 * H + 16]
    dist = jnp.sqrt(jnp.sum(d3 * d3, axis=1, keepdims=True))
    w1b = pr[0:1, 0:H]
    w2b = pr[1:2, 0:H]
    w1d = pr[2:3, 0:H]
    w2d = pr[3:4, 0:H]
    c1a = pr[4:5, 0:H]
    c2a = pr[5:6, 0:H]
    b1b = pr[6:7, 0:1]
    b2b = pr[6:7, 1:2]
    h1 = a1 + b1 + dist * w1d + c1a
    h2 = a2 + b2 + dist * w2d + c2a
    s1 = h1 * jax.nn.sigmoid(h1)
    s2 = h2 * jax.nn.sigmoid(h2)
    mes1 = jnp.sum(s1 * w1b, axis=1, keepdims=True) + b1b
    mes2 = jnp.sum(s2 * w2b, axis=1, keepdims=True) + b2b
    rmax = 4.5
    t = jnp.clip(dist, 0.0, rmax) / rmax
    t2 = t * t
    t4 = t2 * t2
    t5 = t4 * t
    t6 = t5 * t
    t7 = t6 * t
    coe = 1.0 - 21.0 * t5 + 35.0 * t6 - 15.0 * t7
    inv = coe / (dist + 1e-6)
    v1 = d3 * (inv * mes1)
    v2 = d3 * (inv * mes2)
    v_ref[...] = jnp.concatenate([v1[:, 0:8], v2[:, 0:8]], axis=1)


def _edge_math(gr, gc, params):
    return pl.pallas_call(
        _edge_body,
        grid=(EPAD // BE,),
        in_specs=[pl.BlockSpec((BE, TW), lambda i: (i, 0)),
                  pl.BlockSpec((BE, TW), lambda i: (i, 0)),
                  pl.BlockSpec((8, 128), lambda i: (0, 0))],
        out_specs=pl.BlockSpec((BE, 16), lambda i: (i, 0)),
        out_shape=jax.ShapeDtypeStruct((EPAD, 16), jnp.float32),
    )(gr, gc, params)


def _sc_scatter(v, col3):
    mesh = plsc.VectorSubcoreMesh(core_axis_name="c", subcore_axis_name="s")
    ZR = NPAD // NS

    @functools.partial(
        pl.kernel, mesh=mesh,
        compiler_params=_SC_PARAMS,
        out_type=jax.ShapeDtypeStruct((NC, NPAD, 16), jnp.float32),
        scratch_types=[pltpu.VMEM((CH, 16), jnp.float32),
                       pltpu.VMEM((CH, 16), jnp.float32),
                       pltpu.VMEM((NCH, CH), jnp.int32),
                       pltpu.VMEM_SHARED((NPAD, 16), jnp.float32),
                       pltpu.SemaphoreType.DMA,
                       pltpu.SemaphoreType.DMA,
                       pltpu.SemaphoreType.DMA,
                       pltpu.SemaphoreType.DMA,
                       pltpu.SemaphoreType.DMA],
    )
    def k(v_hbm, col_hbm, out_hbm, vb0, vb1, idxbuf, acc,
          ls0, ls1, ss0, ss1, isem):
        cid = lax.axis_index("c")
        sid = lax.axis_index("s")
        wid = sid * NC + cid
        base = wid * PERW

        @pl.loop(0, CH)
        def _(i):
            vb0[i, :] = jnp.zeros((16,), jnp.float32)

        @pl.loop(0, ZR // CH)
        def _(j):
            pltpu.sync_copy(vb0, acc.at[pl.ds(sid * ZR + j * CH, CH)])

        pltpu.async_copy(col_hbm.at[wid], idxbuf, isem).wait()
        plsc.subcore_barrier()

        bufs = ((vb0, ls0, ss0), (vb1, ls1, ss1))

        def start_load(j, p):
            vb, ls, _ = bufs[p]
            off = base + j * CH
            pltpu.make_async_copy(v_hbm.at[pl.ds(off, CH)], vb, ls).start()

        def wait_load(j, p):
            vb, ls, _ = bufs[p]
            off = base + j * CH
            pltpu.make_async_copy(v_hbm.at[pl.ds(off, CH)], vb, ls).wait()

        def start_scatter(j, p):
            vb, _, ss = bufs[p]
            pltpu.async_copy(vb, acc.at[idxbuf.at[j]], ss, add=True)

        def wait_scatter(j, p):
            vb, _, ss = bufs[p]
            pltpu.make_async_copy(vb, acc.at[idxbuf.at[j]], ss).wait()

        start_load(0, 0)
        start_load(1, 1)

        @pl.loop(0, NCH // 2)
        def _(i):
            j0 = 2 * i
            for p in range(2):
                j = j0 + p
                wait_load(j, p)
                start_scatter(j, p)
                wait_scatter(j, p)

                @pl.when(j + 2 < NCH)
                def _():
                    start_load(j + 2, p)

        plsc.subcore_barrier()
        pltpu.sync_copy(acc.at[pl.ds(sid * ZR, ZR)],
                        out_hbm.at[cid, pl.ds(sid * ZR, ZR)])

    return k(v, col3)


def _fin_body(p_ref, o_ref):
    p = p_ref[0] + p_ref[1]
    eps = 1e-6
    v1 = p[:, 0:3]
    v2 = p[:, 8:11]
    v1n = jnp.sqrt(jnp.sum(v1 * v1, axis=1, keepdims=True))
    one = jnp.ones_like(v1n)
    zero = jnp.zeros_like(v1n)
    default = jnp.concatenate([one, zero, zero], axis=1)
    n1 = jnp.where(v1n > eps, v1 / (v1n + eps), default)
    n2p = v2 - jnp.sum(n1 * v2, axis=1, keepdims=True) * n1
    n2n = jnp.sqrt(jnp.sum(n2p * n2p, axis=1, keepdims=True))
    fb = jnp.concatenate([-n1[:, 1:2], n1[:, 0:1], zero], axis=1)
    fb = fb - jnp.sum(n1 * fb, axis=1, keepdims=True) * n1
    fbn = jnp.sqrt(jnp.sum(fb * fb, axis=1, keepdims=True))
    fb = fb / (fbn + eps)
    n2 = jnp.where(n2n > eps, n2p / (n2n + eps), fb)
    c0 = n1[:, 1:2] * n2[:, 2:3] - n1[:, 2:3] * n2[:, 1:2]
    c1 = n1[:, 2:3] * n2[:, 0:1] - n1[:, 0:1] * n2[:, 2:3]
    c2 = n1[:, 0:1] * n2[:, 1:2] - n1[:, 1:2] * n2[:, 0:1]
    n3 = jnp.concatenate([c0, c1, c2], axis=1)
    n3n = jnp.sqrt(jnp.sum(n3 * n3, axis=1, keepdims=True))
    n3 = n3 / (n3n + eps)
    pad7 = jnp.zeros((p.shape[0], 7), jnp.float32)
    o_ref[...] = jnp.concatenate(
        [n1[:, 0:1], n2[:, 0:1], n3[:, 0:1],
         n1[:, 1:2], n2[:, 1:2], n3[:, 1:2],
         n1[:, 2:3], n2[:, 2:3], n3[:, 2:3], pad7], axis=1)


def _finalize(p):
    blk = 1024
    return pl.pallas_call(
        _fin_body,
        grid=(NPAD // blk,),
        in_specs=[pl.BlockSpec((NC, blk, 16), lambda i: (0, i, 0))],
        out_specs=pl.BlockSpec((blk, 16), lambda i: (i, 0)),
        out_shape=jax.ShapeDtypeStruct((NPAD, 16), jnp.float32),
    )(p)


def kernel(x, pos, edge_index, W1a, b1a, W1b, b1b, W2a, b2a, W2b, b2b):
    Wr = jnp.concatenate([W1a[:F], W2a[:F]], axis=1)
    Wc = jnp.concatenate([W1a[F:2 * F], W2a[F:2 * F]], axis=1)
    params = jnp.zeros((8, 128), jnp.float32)
    params = params.at[0, :H].set(W1b[:, 0])
    params = params.at[1, :H].set(W2b[:, 0])
    params = params.at[2, :H].set(W1a[2 * F])
    params = params.at[3, :H].set(W2a[2 * F])
    params = params.at[4, :H].set(b1a)
    params = params.at[5, :H].set(b2a)
    params = params.at[6, 0].set(b1b[0])
    params = params.at[6, 1].set(b2b[0])
    padlen = EPAD - E
    row = jnp.concatenate([edge_index[0], jnp.zeros((padlen,), jnp.int32)])
    col = jnp.concatenate([edge_index[1], jnp.zeros((padlen,), jnp.int32)])
    row3 = row.reshape(NW, NCH, CH)
    col3 = col.reshape(NW, NCH, CH)
    tr, tc_ = _node_tables(x, pos, Wr, Wc)
    gr, gc = _sc_gather(tr, tc_, row3, col3)
    v = _edge_math(gr, gc, params)
    p = _sc_scatter(v, col3)
    o = _finalize(p)
    return o[:N, :9].reshape(N, 3, 3)


# speedup vs baseline: 2.5294x; 1.0729x over previous
import functools

import jax
import jax.numpy as jnp
from jax import lax
from jax.experimental import pallas as pl
from jax.experimental.pallas import tpu as pltpu
from jax.experimental.pallas import tpu_sc as plsc

N = 10000
E = 320000
F = 128
H = 64
NC, NS = 2, 16
NW = NC * NS
CH = 128
EPAD = 327680
PERW = EPAD // NW
NCH = PERW // CH
NPAD = 10240
TW = 144
BE = 4096


def _tables_body(x_ref, pos_ref, wr_ref, wc_ref, tr_ref, tc_ref):
    x = x_ref[...]
    p3 = pos_ref[...]
    pad = jnp.zeros((x.shape[0], TW - F - 3), jnp.float32)
    tr_ref[...] = jnp.concatenate(
        [jnp.dot(x, wr_ref[...], preferred_element_type=jnp.float32), p3, pad],
        axis=1)
    tc_ref[...] = jnp.concatenate(
        [jnp.dot(x, wc_ref[...], preferred_element_type=jnp.float32), p3, pad],
        axis=1)


def _node_tables(x, pos, Wr, Wc):
    blk = 1000
    return pl.pallas_call(
        _tables_body,
        grid=(N // blk,),
        in_specs=[pl.BlockSpec((blk, F), lambda i: (i, 0)),
                  pl.BlockSpec((blk, 3), lambda i: (i, 0)),
                  pl.BlockSpec((F, F), lambda i: (0, 0)),
                  pl.BlockSpec((F, F), lambda i: (0, 0))],
        out_specs=[pl.BlockSpec((blk, TW), lambda i: (i, 0)),
                   pl.BlockSpec((blk, TW), lambda i: (i, 0))],
        out_shape=[jax.ShapeDtypeStruct((N, TW), jnp.float32)] * 2,
    )(x, pos, Wr, Wc)


_SC_PARAMS = pltpu.CompilerParams(use_tc_tiling_on_sc=False)


NCHT = 2 * NCH
NCH0 = 112
NCH1 = NCHT - NCH0


def _sc_gather(tr, tc_, row3, col3):
    mesh = plsc.VectorSubcoreMesh(core_axis_name="c", subcore_axis_name="s")

    @functools.partial(
        pl.kernel, mesh=mesh,
        compiler_params=_SC_PARAMS,
        out_type=[jax.ShapeDtypeStruct((EPAD, TW), jnp.float32)] * 2,
        scratch_types=[pltpu.VMEM((NCH0, CH), jnp.int32),
                       pltpu.VMEM((NCH0, CH), jnp.int32),
                       pltpu.VMEM((CH, TW), jnp.float32),
                       pltpu.VMEM((CH, TW), jnp.float32),
                       pltpu.VMEM((CH, TW), jnp.float32),
                       pltpu.VMEM((CH, TW), jnp.float32),
                       pltpu.SemaphoreType.DMA,
                       pltpu.SemaphoreType.DMA,
                       pltpu.SemaphoreType.DMA,
                       pltpu.SemaphoreType.DMA,
                       pltpu.SemaphoreType.DMA,
                       pltpu.SemaphoreType.DMA,
                       pltpu.SemaphoreType.DMA,
                       pltpu.SemaphoreType.DMA,
                       pltpu.SemaphoreType.DMA],
    )
    def k(tr_hbm, tc_hbm, row_hbm, col_hbm, gr_hbm, gc_hbm,
          idxr, idxc, br0, bc0, br1, bc1,
          gsr0, gsc0, gsr1, gsc1, wsr0, wsc0, wsr1, wsc1, isem):
        cid = lax.axis_index("c")
        sid = lax.axis_index("s")
        pair_base = sid * (NCHT * CH)
        bufs = ((br0, bc0, gsr0, gsc0, wsr0, wsc0),
                (br1, bc1, gsr1, gsc1, wsr1, wsc1))

        def pipeline(nch, chunk0, ebase):
            pltpu.async_copy(row_hbm.at[sid, pl.ds(chunk0, nch)],
                             idxr.at[pl.ds(0, nch)], isem).wait()
            pltpu.async_copy(col_hbm.at[sid, pl.ds(chunk0, nch)],
                             idxc.at[pl.ds(0, nch)], isem).wait()

            def start_gather(j, p):
                br, bc, gsr, gsc, _, _ = bufs[p]
                pltpu.make_async_copy(tr_hbm.at[idxr.at[j]], br, gsr).start()
                pltpu.make_async_copy(tc_hbm.at[idxc.at[j]], bc, gsc).start()

            def wait_gather(j, p):
                br, bc, gsr, gsc, _, _ = bufs[p]
                pltpu.make_async_copy(tr_hbm.at[idxr.at[j]], br, gsr).wait()
                pltpu.make_async_copy(tc_hbm.at[idxc.at[j]], bc, gsc).wait()

            def start_write(j, p):
                br, bc, _, _, wsr, wsc = bufs[p]
                off = ebase + j * CH
                pltpu.make_async_copy(br, gr_hbm.at[pl.ds(off, CH)],
                                      wsr).start()
                pltpu.make_async_copy(bc, gc_hbm.at[pl.ds(off, CH)],
                                      wsc).start()

            def wait_write(j, p):
                br, bc, _, _, wsr, wsc = bufs[p]
                off = ebase + j * CH
                pltpu.make_async_copy(br, gr_hbm.at[pl.ds(off, CH)],
                                      wsr).wait()
                pltpu.make_async_copy(bc, gc_hbm.at[pl.ds(off, CH)],
                                      wsc).wait()

            start_gather(0, 0)
            start_gather(1, 1)
            wait_gather(0, 0)
            start_write(0, 0)
            wait_gather(1, 1)
            start_write(1, 1)

            @pl.loop(1, nch // 2)
            def _(i):
                j0 = 2 * i
                for p in range(2):
                    j = j0 + p
                    wait_write(j, p)
                    start_gather(j, p)
                    wait_gather(j, p)
                    start_write(j, p)

            wait_write(nch - 2, 0)
            wait_write(nch - 1, 1)

        @pl.when(cid == 0)
        def _():
            pipeline(NCH0, 0, pair_base)

        @pl.when(cid == 1)
        def _():
            pipeline(NCH1, NCH0, pair_base + NCH0 * CH)

    return k(tr, tc_, row3, col3)


def _edge_body(gr_ref, gc_ref, on_ref, wd_ref, wm_ref, ba_ref, bb_ref, v_ref):
    gr = gr_ref[...]
    gc = gc_ref[...]
    d3 = gr[:, 2 * H:2 * H + 16] - gc[:, 2 * H:2 * H + 16]
    dist2 = jnp.dot(d3 * d3, on_ref[...],
                    preferred_element_type=jnp.float32)
    dist16 = jnp.sqrt(dist2)
    distwd = jnp.dot(dist16, wd_ref[...],
                     preferred_element_type=jnp.float32)
    h = gr[:, 0:2 * H] + gc[:, 0:2 * H] + distwd + ba_ref[0:1, :]
    s = h * jax.nn.sigmoid(h)
    mes = jnp.dot(s, wm_ref[...],
                  preferred_element_type=jnp.float32) + bb_ref[0:1, :]
    rmax = 4.5
    t = jnp.clip(dist16, 0.0, rmax) / rmax
    t2 = t * t
    t4 = t2 * t2
    t5 = t4 * t
    t6 = t5 * t
    t7 = t6 * t
    coe = 1.0 - 21.0 * t5 + 35.0 * t6 - 15.0 * t7
    fac = (coe / (dist16 + 1e-6)) * mes
    d3pair = jnp.concatenate([d3[:, 0:8], d3[:, 0:8]], axis=1)
    v_ref[...] = d3pair * fac


def _edge_math(gr, gc, ones16, WD, WM, BA, BB):
    return pl.pallas_call(
        _edge_body,
        grid=(EPAD // BE,),
        in_specs=[pl.BlockSpec((BE, TW), lambda i: (i, 0)),
                  pl.BlockSpec((BE, TW), lambda i: (i, 0)),
                  pl.BlockSpec((16, 16), lambda i: (0, 0)),
                  pl.BlockSpec((16, 128), lambda i: (0, 0)),
                  pl.BlockSpec((128, 16), lambda i: (0, 0)),
                  pl.BlockSpec((8, 128), lambda i: (0, 0)),
                  pl.BlockSpec((8, 16), lambda i: (0, 0))],
        out_specs=pl.BlockSpec((BE, 16), lambda i: (i, 0)),
        out_shape=jax.ShapeDtypeStruct((EPAD, 16), jnp.float32),
    )(gr, gc, ones16, WD, WM, BA, BB)


def _sc_scatter(v, col3):
    mesh = plsc.VectorSubcoreMesh(core_axis_name="c", subcore_axis_name="s")
    ZR = NPAD // NS

    @functools.partial(
        pl.kernel, mesh=mesh,
        compiler_params=_SC_PARAMS,
        out_type=jax.ShapeDtypeStruct((NC, NPAD, 16), jnp.float32),
        scratch_types=[pltpu.VMEM((CH, 16), jnp.float32),
                       pltpu.VMEM((CH, 16), jnp.float32),
                       pltpu.VMEM((NCH, CH), jnp.int32),
                       pltpu.VMEM_SHARED((NPAD, 16), jnp.float32),
                       pltpu.SemaphoreType.DMA,
                       pltpu.SemaphoreType.DMA,
                       pltpu.SemaphoreType.DMA,
                       pltpu.SemaphoreType.DMA,
                       pltpu.SemaphoreType.DMA],
    )
    def k(v_hbm, col_hbm, out_hbm, vb0, vb1, idxbuf, acc,
          ls0, ls1, ss0, ss1, isem):
        cid = lax.axis_index("c")
        sid = lax.axis_index("s")
        wid = sid * NC + cid
        base = wid * PERW

        @pl.loop(0, CH)
        def _(i):
            vb0[i, :] = jnp.zeros((16,), jnp.float32)

        @pl.loop(0, ZR // CH)
        def _(j):
            pltpu.sync_copy(vb0, acc.at[pl.ds(sid * ZR + j * CH, CH)])

        pltpu.async_copy(col_hbm.at[wid], idxbuf, isem).wait()
        plsc.subcore_barrier()

        bufs = ((vb0, ls0, ss0), (vb1, ls1, ss1))

        def start_load(j, p):
            vb, ls, _ = bufs[p]
            off = base + j * CH
            pltpu.make_async_copy(v_hbm.at[pl.ds(off, CH)], vb, ls).start()

        def wait_load(j, p):
            vb, ls, _ = bufs[p]
            off = base + j * CH
            pltpu.make_async_copy(v_hbm.at[pl.ds(off, CH)], vb, ls).wait()

        def start_scatter(j, p):
            vb, _, ss = bufs[p]
            pltpu.async_copy(vb, acc.at[idxbuf.at[j]], ss, add=True)

        def wait_scatter(j, p):
            vb, _, ss = bufs[p]
            pltpu.make_async_copy(vb, acc.at[idxbuf.at[j]], ss).wait()

        start_load(0, 0)
        start_load(1, 1)

        @pl.loop(0, NCH // 2)
        def _(i):
            j0 = 2 * i
            for p in range(2):
                j = j0 + p
                wait_load(j, p)
                start_scatter(j, p)
                wait_scatter(j, p)

                @pl.when(j + 2 < NCH)
                def _():
                    start_load(j + 2, p)

        plsc.subcore_barrier()
        pltpu.sync_copy(acc.at[pl.ds(sid * ZR, ZR)],
                        out_hbm.at[cid, pl.ds(sid * ZR, ZR)])

    return k(v, col3)


def _fin_body(p_ref, o_ref):
    p = p_ref[0] + p_ref[1]
    eps = 1e-6
    v1 = p[:, 0:3]
    v2 = p[:, 8:11]
    v1n = jnp.sqrt(jnp.sum(v1 * v1, axis=1, keepdims=True))
    one = jnp.ones_like(v1n)
    zero = jnp.zeros_like(v1n)
    default = jnp.concatenate([one, zero, zero], axis=1)
    n1 = jnp.where(v1n > eps, v1 / (v1n + eps), default)
    n2p = v2 - jnp.sum(n1 * v2, axis=1, keepdims=True) * n1
    n2n = jnp.sqrt(jnp.sum(n2p * n2p, axis=1, keepdims=True))
    fb = jnp.concatenate([-n1[:, 1:2], n1[:, 0:1], zero], axis=1)
    fb = fb - jnp.sum(n1 * fb, axis=1, keepdims=True) * n1
    fbn = jnp.sqrt(jnp.sum(fb * fb, axis=1, keepdims=True))
    fb = fb / (fbn + eps)
    n2 = jnp.where(n2n > eps, n2p / (n2n + eps), fb)
    c0 = n1[:, 1:2] * n2[:, 2:3] - n1[:, 2:3] * n2[:, 1:2]
    c1 = n1[:, 2:3] * n2[:, 0:1] - n1[:, 0:1] * n2[:, 2:3]
    c2 = n1[:, 0:1] * n2[:, 1:2] - n1[:, 1:2] * n2[:, 0:1]
    n3 = jnp.concatenate([c0, c1, c2], axis=1)
    n3n = jnp.sqrt(jnp.sum(n3 * n3, axis=1, keepdims=True))
    n3 = n3 / (n3n + eps)
    pad7 = jnp.zeros((p.shape[0], 7), jnp.float32)
    o_ref[...] = jnp.concatenate(
        [n1[:, 0:1], n2[:, 0:1], n3[:, 0:1],
         n1[:, 1:2], n2[:, 1:2], n3[:, 1:2],
         n1[:, 2:3], n2[:, 2:3], n3[:, 2:3], pad7], axis=1)


def _finalize(p):
    blk = 1024
    return pl.pallas_call(
        _fin_body,
        grid=(NPAD // blk,),
        in_specs=[pl.BlockSpec((NC, blk, 16), lambda i: (0, i, 0))],
        out_specs=pl.BlockSpec((blk, 16), lambda i: (i, 0)),
        out_shape=jax.ShapeDtypeStruct((NPAD, 16), jnp.float32),
    )(p)


def kernel(x, pos, edge_index, W1a, b1a, W1b, b1b, W2a, b2a, W2b, b2b):
    Wr = jnp.concatenate([W1a[:F], W2a[:F]], axis=1)
    Wc = jnp.concatenate([W1a[F:2 * F], W2a[F:2 * F]], axis=1)
    ones16 = jnp.ones((16, 16), jnp.float32)
    wd128 = jnp.concatenate([W1a[2 * F], W2a[2 * F]])
    WD = jnp.tile(wd128[None, :] / 16.0, (16, 1))
    WM = jnp.zeros((128, 16), jnp.float32)
    WM = WM.at[0:H, 0:8].set(jnp.tile(W1b, (1, 8)))
    WM = WM.at[H:2 * H, 8:16].set(jnp.tile(W2b, (1, 8)))
    BA = jnp.zeros((8, 128), jnp.float32)
    BA = BA.at[0, 0:H].set(b1a)
    BA = BA.at[0, H:2 * H].set(b2a)
    BB = jnp.zeros((8, 16), jnp.float32)
    BB = BB.at[0, 0:8].set(b1b[0])
    BB = BB.at[0, 8:16].set(b2b[0])
    padlen = EPAD - E
    row = jnp.concatenate([edge_index[0], jnp.zeros((padlen,), jnp.int32)])
    col = jnp.concatenate([edge_index[1], jnp.zeros((padlen,), jnp.int32)])
    rowg = row.reshape(NS, NCHT, CH)
    colg = col.reshape(NS, NCHT, CH)
    col3 = col.reshape(NW, NCH, CH)
    tr, tc_ = _node_tables(x, pos, Wr, Wc)
    gr, gc = _sc_gather(tr, tc_, rowg, colg)
    v = _edge_math(gr, gc, ones16, WD, WM, BA, BB)
    p = _sc_scatter(v, col3)
    o = _finalize(p)
    return o[:N, :9].reshape(N, 3, 3)


# speedup vs baseline: 3.5797x; 1.4152x over previous
import functools

import jax
import jax.numpy as jnp
from jax import lax
from jax.experimental import pallas as pl
from jax.experimental.pallas import tpu as pltpu
from jax.experimental.pallas import tpu_sc as plsc

N = 10000
E = 320000
F = 128
H = 64
NC, NS = 2, 16
NW = NC * NS
CH = 128
EPAD = 327680
PERW = EPAD // NW
NCH = PERW // CH
NPAD = 10240
TW = 144
BE = 4096


def _tables_body(x_ref, pos_ref, wr_ref, wc_ref, tr_ref, tc_ref):
    x = x_ref[...]
    p3 = pos_ref[...]
    pad = jnp.zeros((x.shape[0], TW - F - 3), jnp.float32)
    tr_ref[...] = jnp.concatenate(
        [jnp.dot(x, wr_ref[...], preferred_element_type=jnp.float32), p3, pad],
        axis=1)
    tc_ref[...] = jnp.concatenate(
        [jnp.dot(x, wc_ref[...], preferred_element_type=jnp.float32), p3, pad],
        axis=1)


def _node_tables(x, pos, Wr, Wc):
    blk = 1000
    return pl.pallas_call(
        _tables_body,
        grid=(N // blk,),
        in_specs=[pl.BlockSpec((blk, F), lambda i: (i, 0)),
                  pl.BlockSpec((blk, 3), lambda i: (i, 0)),
                  pl.BlockSpec((F, F), lambda i: (0, 0)),
                  pl.BlockSpec((F, F), lambda i: (0, 0))],
        out_specs=[pl.BlockSpec((blk, TW), lambda i: (i, 0)),
                   pl.BlockSpec((blk, TW), lambda i: (i, 0))],
        out_shape=[jax.ShapeDtypeStruct((N, TW), jnp.float32)] * 2,
    )(x, pos, Wr, Wc)


_SC_PARAMS = pltpu.CompilerParams(use_tc_tiling_on_sc=False)


NCHT = 2 * NCH
NCH0 = 112
NCH1 = NCHT - NCH0


def _sc_gather(tr, tc_, row3, col3):
    mesh = plsc.VectorSubcoreMesh(core_axis_name="c", subcore_axis_name="s")

    @functools.partial(
        pl.kernel, mesh=mesh,
        compiler_params=_SC_PARAMS,
        out_type=[jax.ShapeDtypeStruct((EPAD, F), jnp.float32),
                  jax.ShapeDtypeStruct((EPAD, F), jnp.float32),
                  jax.ShapeDtypeStruct((EPAD, 32), jnp.float32)],
        scratch_types=[pltpu.VMEM((NCH0, CH), jnp.int32),
                       pltpu.VMEM((NCH0, CH), jnp.int32),
                       pltpu.VMEM((CH, TW), jnp.float32),
                       pltpu.VMEM((CH, TW), jnp.float32),
                       pltpu.VMEM((CH, TW), jnp.float32),
                       pltpu.VMEM((CH, TW), jnp.float32),
                       pltpu.SemaphoreType.DMA,
                       pltpu.SemaphoreType.DMA,
                       pltpu.SemaphoreType.DMA,
                       pltpu.SemaphoreType.DMA,
                       pltpu.SemaphoreType.DMA,
                       pltpu.SemaphoreType.DMA,
                       pltpu.SemaphoreType.DMA,
                       pltpu.SemaphoreType.DMA,
                       pltpu.SemaphoreType.DMA,
                       pltpu.SemaphoreType.DMA,
                       pltpu.SemaphoreType.DMA],
    )
    def k(tr_hbm, tc_hbm, row_hbm, col_hbm, gr_hbm, gc_hbm, q_hbm,
          idxr, idxc, br0, bc0, br1, bc1,
          gsr0, gsc0, gsr1, gsc1, wsr0, wsc0, wsq0, wsr1, wsc1, wsq1, isem):
        cid = lax.axis_index("c")
        sid = lax.axis_index("s")
        pair_base = sid * (NCHT * CH)
        bufs = ((br0, bc0, gsr0, gsc0, wsr0, wsc0, wsq0),
                (br1, bc1, gsr1, gsc1, wsr1, wsc1, wsq1))

        def pipeline(nch, chunk0, ebase):
            pltpu.async_copy(row_hbm.at[sid, pl.ds(chunk0, nch)],
                             idxr.at[pl.ds(0, nch)], isem).wait()
            pltpu.async_copy(col_hbm.at[sid, pl.ds(chunk0, nch)],
                             idxc.at[pl.ds(0, nch)], isem).wait()

            def start_gather(j, p):
                br, bc, gsr, gsc = bufs[p][:4]
                pltpu.make_async_copy(tr_hbm.at[idxr.at[j]], br, gsr).start()
                pltpu.make_async_copy(tc_hbm.at[idxc.at[j]], bc, gsc).start()

            def wait_gather(j, p):
                br, bc, gsr, gsc = bufs[p][:4]
                pltpu.make_async_copy(tr_hbm.at[idxr.at[j]], br, gsr).wait()
                pltpu.make_async_copy(tc_hbm.at[idxc.at[j]], bc, gsc).wait()

            def _write_copies(j, p):
                br, bc, _, _, wsr, wsc, wsq = bufs[p]
                off = ebase + j * CH
                rows = pl.ds(off, CH)
                return (
                    pltpu.make_async_copy(
                        br.at[:, pl.ds(0, F)], gr_hbm.at[rows], wsr),
                    pltpu.make_async_copy(
                        bc.at[:, pl.ds(0, F)], gc_hbm.at[rows], wsc),
                    pltpu.make_async_copy(
                        br.at[:, pl.ds(F, 16)],
                        q_hbm.at[rows, pl.ds(0, 16)], wsq),
                    pltpu.make_async_copy(
                        bc.at[:, pl.ds(F, 16)],
                        q_hbm.at[rows, pl.ds(16, 16)], wsq),
                )

            def start_write(j, p):
                for c in _write_copies(j, p):
                    c.start()

            def wait_write(j, p):
                for c in _write_copies(j, p):
                    c.wait()

            start_gather(0, 0)
            start_gather(1, 1)
            wait_gather(0, 0)
            start_write(0, 0)
            wait_gather(1, 1)
            start_write(1, 1)

            @pl.loop(1, nch // 2)
            def _(i):
                j0 = 2 * i
                for p in range(2):
                    j = j0 + p
                    wait_write(j, p)
                    start_gather(j, p)
                    wait_gather(j, p)
                    start_write(j, p)

            wait_write(nch - 2, 0)
            wait_write(nch - 1, 1)

        @pl.when(cid == 0)
        def _():
            pipeline(NCH0, 0, pair_base)

        @pl.when(cid == 1)
        def _():
            pipeline(NCH1, NCH0, pair_base + NCH0 * CH)

    return k(tr, tc_, row3, col3)


def _edge_body(gr_ref, gc_ref, q_ref, on_ref, wd_ref, wm_ref, ba_ref, bb_ref,
               v_ref):
    q = q_ref[...]
    d3 = q[:, 0:16] - q[:, 16:32]
    dist2 = jnp.dot(d3 * d3, on_ref[...],
                    preferred_element_type=jnp.float32)
    dist16 = jnp.sqrt(dist2)
    distwd = jnp.dot(dist16, wd_ref[...],
                     preferred_element_type=jnp.float32)
    h = gr_ref[...] + gc_ref[...] + distwd + ba_ref[0:1, :]
    s = h * jax.nn.sigmoid(h)
    mes = jnp.dot(s, wm_ref[...],
                  preferred_element_type=jnp.float32) + bb_ref[0:1, :]
    rmax = 4.5
    t = jnp.clip(dist16, 0.0, rmax) / rmax
    t2 = t * t
    t4 = t2 * t2
    t5 = t4 * t
    t6 = t5 * t
    t7 = t6 * t
    coe = 1.0 - 21.0 * t5 + 35.0 * t6 - 15.0 * t7
    fac = (coe / (dist16 + 1e-6)) * mes
    d3pair = jnp.concatenate([d3[:, 0:8], d3[:, 0:8]], axis=1)
    v_ref[...] = d3pair * fac


def _edge_math(gr, gc, q, ones16, WD, WM, BA, BB):
    return pl.pallas_call(
        _edge_body,
        grid=(EPAD // BE,),
        in_specs=[pl.BlockSpec((BE, F), lambda i: (i, 0)),
                  pl.BlockSpec((BE, F), lambda i: (i, 0)),
                  pl.BlockSpec((BE, 32), lambda i: (i, 0)),
                  pl.BlockSpec((16, 16), lambda i: (0, 0)),
                  pl.BlockSpec((16, 128), lambda i: (0, 0)),
                  pl.BlockSpec((128, 16), lambda i: (0, 0)),
                  pl.BlockSpec((8, 128), lambda i: (0, 0)),
                  pl.BlockSpec((8, 16), lambda i: (0, 0))],
        out_specs=pl.BlockSpec((BE, 16), lambda i: (i, 0)),
        out_shape=jax.ShapeDtypeStruct((EPAD, 16), jnp.float32),
    )(gr, gc, q, ones16, WD, WM, BA, BB)


def _sc_scatter(v, col3):
    mesh = plsc.VectorSubcoreMesh(core_axis_name="c", subcore_axis_name="s")
    ZR = NPAD // NS

    @functools.partial(
        pl.kernel, mesh=mesh,
        compiler_params=_SC_PARAMS,
        out_type=jax.ShapeDtypeStruct((NC, NPAD, 16), jnp.float32),
        scratch_types=[pltpu.VMEM((CH, 16), jnp.float32),
                       pltpu.VMEM((CH, 16), jnp.float32),
                       pltpu.VMEM((NCH, CH), jnp.int32),
                       pltpu.VMEM_SHARED((NPAD, 16), jnp.float32),
                       pltpu.SemaphoreType.DMA,
                       pltpu.SemaphoreType.DMA,
                       pltpu.SemaphoreType.DMA,
                       pltpu.SemaphoreType.DMA,
                       pltpu.SemaphoreType.DMA],
    )
    def k(v_hbm, col_hbm, out_hbm, vb0, vb1, idxbuf, acc,
          ls0, ls1, ss0, ss1, isem):
        cid = lax.axis_index("c")
        sid = lax.axis_index("s")
        wid = sid * NC + cid
        base = wid * PERW

        @pl.loop(0, CH)
        def _(i):
            vb0[i, :] = jnp.zeros((16,), jnp.float32)

        @pl.loop(0, ZR // CH)
        def _(j):
            pltpu.sync_copy(vb0, acc.at[pl.ds(sid * ZR + j * CH, CH)])

        pltpu.async_copy(col_hbm.at[wid], idxbuf, isem).wait()
        plsc.subcore_barrier()

        bufs = ((vb0, ls0, ss0), (vb1, ls1, ss1))

        def start_load(j, p):
            vb, ls, _ = bufs[p]
            off = base + j * CH
            pltpu.make_async_copy(v_hbm.at[pl.ds(off, CH)], vb, ls).start()

        def wait_load(j, p):
            vb, ls, _ = bufs[p]
            off = base + j * CH
            pltpu.make_async_copy(v_hbm.at[pl.ds(off, CH)], vb, ls).wait()

        def start_scatter(j, p):
            vb, _, ss = bufs[p]
            pltpu.async_copy(vb, acc.at[idxbuf.at[j]], ss, add=True)

        def wait_scatter(j, p):
            vb, _, ss = bufs[p]
            pltpu.make_async_copy(vb, acc.at[idxbuf.at[j]], ss).wait()

        start_load(0, 0)
        start_load(1, 1)

        @pl.loop(0, NCH // 2)
        def _(i):
            j0 = 2 * i
            for p in range(2):
                j = j0 + p
                wait_load(j, p)
                start_scatter(j, p)
                wait_scatter(j, p)

                @pl.when(j + 2 < NCH)
                def _():
                    start_load(j + 2, p)

        plsc.subcore_barrier()
        pltpu.sync_copy(acc.at[pl.ds(sid * ZR, ZR)],
                        out_hbm.at[cid, pl.ds(sid * ZR, ZR)])

    return k(v, col3)


def _fin_body(p_ref, o_ref):
    p = p_ref[0] + p_ref[1]
    eps = 1e-6
    v1 = p[:, 0:3]
    v2 = p[:, 8:11]
    v1n = jnp.sqrt(jnp.sum(v1 * v1, axis=1, keepdims=True))
    one = jnp.ones_like(v1n)
    zero = jnp.zeros_like(v1n)
    default = jnp.concatenate([one, zero, zero], axis=1)
    n1 = jnp.where(v1n > eps, v1 / (v1n + eps), default)
    n2p = v2 - jnp.sum(n1 * v2, axis=1, keepdims=True) * n1
    n2n = jnp.sqrt(jnp.sum(n2p * n2p, axis=1, keepdims=True))
    fb = jnp.concatenate([-n1[:, 1:2], n1[:, 0:1], zero], axis=1)
    fb = fb - jnp.sum(n1 * fb, axis=1, keepdims=True) * n1
    fbn = jnp.sqrt(jnp.sum(fb * fb, axis=1, keepdims=True))
    fb = fb / (fbn + eps)
    n2 = jnp.where(n2n > eps, n2p / (n2n + eps), fb)
    c0 = n1[:, 1:2] * n2[:, 2:3] - n1[:, 2:3] * n2[:, 1:2]
    c1 = n1[:, 2:3] * n2[:, 0:1] - n1[:, 0:1] * n2[:, 2:3]
    c2 = n1[:, 0:1] * n2[:, 1:2] - n1[:, 1:2] * n2[:, 0:1]
    n3 = jnp.concatenate([c0, c1, c2], axis=1)
    n3n = jnp.sqrt(jnp.sum(n3 * n3, axis=1, keepdims=True))
    n3 = n3 / (n3n + eps)
    pad7 = jnp.zeros((p.shape[0], 7), jnp.float32)
    o_ref[...] = jnp.concatenate(
        [n1[:, 0:1], n2[:, 0:1], n3[:, 0:1],
         n1[:, 1:2], n2[:, 1:2], n3[:, 1:2],
         n1[:, 2:3], n2[:, 2:3], n3[:, 2:3], pad7], axis=1)


def _finalize(p):
    blk = 1024
    return pl.pallas_call(
        _fin_body,
        grid=(NPAD // blk,),
        in_specs=[pl.BlockSpec((NC, blk, 16), lambda i: (0, i, 0))],
        out_specs=pl.BlockSpec((blk, 16), lambda i: (i, 0)),
        out_shape=jax.ShapeDtypeStruct((NPAD, 16), jnp.float32),
    )(p)


def kernel(x, pos, edge_index, W1a, b1a, W1b, b1b, W2a, b2a, W2b, b2b):
    Wr = jnp.concatenate([W1a[:F], W2a[:F]], axis=1)
    Wc = jnp.concatenate([W1a[F:2 * F], W2a[F:2 * F]], axis=1)
    ones16 = jnp.ones((16, 16), jnp.float32)
    wd128 = jnp.concatenate([W1a[2 * F], W2a[2 * F]])
    WD = jnp.tile(wd128[None, :] / 16.0, (16, 1))
    WM = jnp.zeros((128, 16), jnp.float32)
    WM = WM.at[0:H, 0:8].set(jnp.tile(W1b, (1, 8)))
    WM = WM.at[H:2 * H, 8:16].set(jnp.tile(W2b, (1, 8)))
    BA = jnp.zeros((8, 128), jnp.float32)
    BA = BA.at[0, 0:H].set(b1a)
    BA = BA.at[0, H:2 * H].set(b2a)
    BB = jnp.zeros((8, 16), jnp.float32)
    BB = BB.at[0, 0:8].set(b1b[0])
    BB = BB.at[0, 8:16].set(b2b[0])
    padlen = EPAD - E
    row = jnp.concatenate([edge_index[0], jnp.zeros((padlen,), jnp.int32)])
    col = jnp.concatenate([edge_index[1], jnp.zeros((padlen,), jnp.int32)])
    rowg = row.reshape(NS, NCHT, CH)
    colg = col.reshape(NS, NCHT, CH)
    col3 = col.reshape(NW, NCH, CH)
    tr, tc_ = _node_tables(x, pos, Wr, Wc)
    gr, gc, q = _sc_gather(tr, tc_, rowg, colg)
    v = _edge_math(gr, gc, q, ones16, WD, WM, BA, BB)
    p = _sc_scatter(v, col3)
    o = _finalize(p)
    return o[:N, :9].reshape(N, 3, 3)


# speedup vs baseline: 4.0677x; 1.1363x over previous
import functools

import jax
import jax.numpy as jnp
from jax import lax
from jax.experimental import pallas as pl
from jax.experimental.pallas import tpu as pltpu
from jax.experimental.pallas import tpu_sc as plsc

N = 10000
E = 320000
F = 128
H = 64
NC, NS = 2, 16
NW = NC * NS
CH = 128
EPAD = 327680
PERW = EPAD // NW
NCH = PERW // CH
NPAD = 10240
TW = 144
BE = 4096


def _tables_body(x_ref, wr_ref, wc_ref, tr_ref, tc_ref):
    x = x_ref[...]
    tr_ref[...] = jnp.dot(x, wr_ref[...], preferred_element_type=jnp.float32)
    tc_ref[...] = jnp.dot(x, wc_ref[...], preferred_element_type=jnp.float32)


def _node_tables(x, Wr, Wc):
    blk = 1000
    return pl.pallas_call(
        _tables_body,
        grid=(N // blk,),
        in_specs=[pl.BlockSpec((blk, F), lambda i: (i, 0)),
                  pl.BlockSpec((F, F), lambda i: (0, 0)),
                  pl.BlockSpec((F, F), lambda i: (0, 0))],
        out_specs=[pl.BlockSpec((blk, F), lambda i: (i, 0)),
                   pl.BlockSpec((blk, F), lambda i: (i, 0))],
        out_shape=[jax.ShapeDtypeStruct((N, F), jnp.float32)] * 2,
    )(x, Wr, Wc)


_SC_PARAMS = pltpu.CompilerParams(use_tc_tiling_on_sc=False)


NCHT = 2 * NCH
NCH0 = 112
NCH1 = NCHT - NCH0


def _sc_gather(tr, tc_, p16, row3, col3):
    mesh = plsc.VectorSubcoreMesh(core_axis_name="c", subcore_axis_name="s")

    @functools.partial(
        pl.kernel, mesh=mesh,
        compiler_params=_SC_PARAMS,
        out_type=[jax.ShapeDtypeStruct((EPAD, F), jnp.float32),
                  jax.ShapeDtypeStruct((EPAD, F), jnp.float32),
                  jax.ShapeDtypeStruct((EPAD, 32), jnp.float32)],
        scratch_types=[pltpu.VMEM((NCH0, CH), jnp.int32),
                       pltpu.VMEM((NCH0, CH), jnp.int32),
                       pltpu.VMEM((CH, F), jnp.float32),
                       pltpu.VMEM((CH, F), jnp.float32),
                       pltpu.VMEM((CH, 16), jnp.float32),
                       pltpu.VMEM((CH, 16), jnp.float32),
                       pltpu.VMEM((CH, F), jnp.float32),
                       pltpu.VMEM((CH, F), jnp.float32),
                       pltpu.VMEM((CH, 16), jnp.float32),
                       pltpu.VMEM((CH, 16), jnp.float32),
                       pltpu.SemaphoreType.DMA,
                       pltpu.SemaphoreType.DMA,
                       pltpu.SemaphoreType.DMA,
                       pltpu.SemaphoreType.DMA,
                       pltpu.SemaphoreType.DMA,
                       pltpu.SemaphoreType.DMA,
                       pltpu.SemaphoreType.DMA,
                       pltpu.SemaphoreType.DMA,
                       pltpu.SemaphoreType.DMA,
                       pltpu.SemaphoreType.DMA,
                       pltpu.SemaphoreType.DMA,
                       pltpu.SemaphoreType.DMA,
                       pltpu.SemaphoreType.DMA],
    )
    def k(tr_hbm, tc_hbm, p16_hbm, row_hbm, col_hbm, gr_hbm, gc_hbm, q_hbm,
          idxr, idxc, br0, bc0, pr0, pc0, br1, bc1, pr1, pc1,
          gsr0, gsc0, gpr0, gpc0, gsr1, gsc1, gpr1, gpc1,
          wsr0, wsq0, wsr1, wsq1, isem):
        cid = lax.axis_index("c")
        sid = lax.axis_index("s")
        pair_base = sid * (NCHT * CH)
        bufs = ((br0, bc0, pr0, pc0, gsr0, gsc0, gpr0, gpc0, wsr0, wsq0),
                (br1, bc1, pr1, pc1, gsr1, gsc1, gpr1, gpc1, wsr1, wsq1))

        def pipeline(nch, chunk0, ebase):
            pltpu.async_copy(row_hbm.at[sid, pl.ds(chunk0, nch)],
                             idxr.at[pl.ds(0, nch)], isem).wait()
            pltpu.async_copy(col_hbm.at[sid, pl.ds(chunk0, nch)],
                             idxc.at[pl.ds(0, nch)], isem).wait()

            def _gather_copies(j, p):
                br, bc, pr, pc, gsr, gsc, gpr, gpc = bufs[p][:8]
                return (
                    pltpu.make_async_copy(tr_hbm.at[idxr.at[j]], br, gsr),
                    pltpu.make_async_copy(tc_hbm.at[idxc.at[j]], bc, gsc),
                    pltpu.make_async_copy(p16_hbm.at[idxr.at[j]], pr, gpr),
                    pltpu.make_async_copy(p16_hbm.at[idxc.at[j]], pc, gpc),
                )

            def start_gather(j, p):
                for c in _gather_copies(j, p):
                    c.start()

            def wait_gather(j, p):
                for c in _gather_copies(j, p):
                    c.wait()

            def _write_copies(j, p):
                br, bc, pr, pc = bufs[p][:4]
                wsr, wsq = bufs[p][8:]
                off = ebase + j * CH
                rows = pl.ds(off, CH)
                return (
                    pltpu.make_async_copy(br, gr_hbm.at[rows], wsr),
                    pltpu.make_async_copy(bc, gc_hbm.at[rows], wsr),
                    pltpu.make_async_copy(
                        pr, q_hbm.at[rows, pl.ds(0, 16)], wsq),
                    pltpu.make_async_copy(
                        pc, q_hbm.at[rows, pl.ds(16, 16)], wsq),
                )

            def start_write(j, p):
                for c in _write_copies(j, p):
                    c.start()

            def wait_write(j, p):
                for c in _write_copies(j, p):
                    c.wait()

            start_gather(0, 0)
            start_gather(1, 1)
            wait_gather(0, 0)
            start_write(0, 0)
            wait_gather(1, 1)
            start_write(1, 1)

            @pl.loop(1, nch // 2)
            def _(i):
                j0 = 2 * i
                for p in range(2):
                    j = j0 + p
                    wait_write(j, p)
                    start_gather(j, p)
                    wait_gather(j, p)
                    start_write(j, p)

            wait_write(nch - 2, 0)
            wait_write(nch - 1, 1)

        @pl.when(cid == 0)
        def _():
            pipeline(NCH0, 0, pair_base)

        @pl.when(cid == 1)
        def _():
            pipeline(NCH1, NCH0, pair_base + NCH0 * CH)

    return k(tr, tc_, p16, row3, col3)


def _edge_body(gr_ref, gc_ref, q_ref, on_ref, wd_ref, wm_ref, ba_ref, bb_ref,
               v_ref):
    q = q_ref[...]
    d3 = q[:, 0:16] - q[:, 16:32]
    dist2 = jnp.dot(d3 * d3, on_ref[...],
                    preferred_element_type=jnp.float32)
    dist16 = jnp.sqrt(dist2)
    distwd = jnp.dot(dist16, wd_ref[...],
                     preferred_element_type=jnp.float32)
    h = gr_ref[...] + gc_ref[...] + distwd + ba_ref[0:1, :]
    s = h * jax.nn.sigmoid(h)
    mes = jnp.dot(s, wm_ref[...],
                  preferred_element_type=jnp.float32) + bb_ref[0:1, :]
    rmax = 4.5
    t = jnp.clip(dist16, 0.0, rmax) / rmax
    t2 = t * t
    t4 = t2 * t2
    t5 = t4 * t
    t6 = t5 * t
    t7 = t6 * t
    coe = 1.0 - 21.0 * t5 + 35.0 * t6 - 15.0 * t7
    fac = (coe / (dist16 + 1e-6)) * mes
    d3pair = jnp.concatenate([d3[:, 0:8], d3[:, 0:8]], axis=1)
    v_ref[...] = d3pair * fac


def _edge_math(gr, gc, q, ones16, WD, WM, BA, BB):
    return pl.pallas_call(
        _edge_body,
        grid=(EPAD // BE,),
        in_specs=[pl.BlockSpec((BE, F), lambda i: (i, 0)),
                  pl.BlockSpec((BE, F), lambda i: (i, 0)),
                  pl.BlockSpec((BE, 32), lambda i: (i, 0)),
                  pl.BlockSpec((16, 16), lambda i: (0, 0)),
                  pl.BlockSpec((16, 128), lambda i: (0, 0)),
                  pl.BlockSpec((128, 16), lambda i: (0, 0)),
                  pl.BlockSpec((8, 128), lambda i: (0, 0)),
                  pl.BlockSpec((8, 16), lambda i: (0, 0))],
        out_specs=pl.BlockSpec((BE, 16), lambda i: (i, 0)),
        out_shape=jax.ShapeDtypeStruct((EPAD, 16), jnp.float32),
    )(gr, gc, q, ones16, WD, WM, BA, BB)


def _sc_scatter(v, col3):
    mesh = plsc.VectorSubcoreMesh(core_axis_name="c", subcore_axis_name="s")
    ZR = NPAD // NS

    @functools.partial(
        pl.kernel, mesh=mesh,
        compiler_params=_SC_PARAMS,
        out_type=jax.ShapeDtypeStruct((NC, NPAD, 16), jnp.float32),
        scratch_types=[pltpu.VMEM((CH, 16), jnp.float32),
                       pltpu.VMEM((CH, 16), jnp.float32),
                       pltpu.VMEM((NCH, CH), jnp.int32),
                       pltpu.VMEM_SHARED((NPAD, 16), jnp.float32),
                       pltpu.SemaphoreType.DMA,
                       pltpu.SemaphoreType.DMA,
                       pltpu.SemaphoreType.DMA,
                       pltpu.SemaphoreType.DMA,
                       pltpu.SemaphoreType.DMA],
    )
    def k(v_hbm, col_hbm, out_hbm, vb0, vb1, idxbuf, acc,
          ls0, ls1, ss0, ss1, isem):
        cid = lax.axis_index("c")
        sid = lax.axis_index("s")
        wid = sid * NC + cid
        base = wid * PERW

        @pl.loop(0, CH)
        def _(i):
            vb0[i, :] = jnp.zeros((16,), jnp.float32)

        @pl.loop(0, ZR // CH)
        def _(j):
            pltpu.sync_copy(vb0, acc.at[pl.ds(sid * ZR + j * CH, CH)])

        pltpu.async_copy(col_hbm.at[wid], idxbuf, isem).wait()
        plsc.subcore_barrier()

        bufs = ((vb0, ls0, ss0), (vb1, ls1, ss1))

        def start_load(j, p):
            vb, ls, _ = bufs[p]
            off = base + j * CH
            pltpu.make_async_copy(v_hbm.at[pl.ds(off, CH)], vb, ls).start()

        def wait_load(j, p):
            vb, ls, _ = bufs[p]
            off = base + j * CH
            pltpu.make_async_copy(v_hbm.at[pl.ds(off, CH)], vb, ls).wait()

        def start_scatter(j, p):
            vb, _, ss = bufs[p]
            pltpu.async_copy(vb, acc.at[idxbuf.at[j]], ss, add=True)

        def wait_scatter(j, p):
            vb, _, ss = bufs[p]
            pltpu.make_async_copy(vb, acc.at[idxbuf.at[j]], ss).wait()

        start_load(0, 0)
        start_load(1, 1)

        @pl.loop(0, NCH // 2)
        def _(i):
            j0 = 2 * i
            for p in range(2):
                j = j0 + p
                wait_load(j, p)
                start_scatter(j, p)
                wait_scatter(j, p)

                @pl.when(j + 2 < NCH)
                def _():
                    start_load(j + 2, p)

        plsc.subcore_barrier()
        pltpu.sync_copy(acc.at[pl.ds(sid * ZR, ZR)],
                        out_hbm.at[cid, pl.ds(sid * ZR, ZR)])

    return k(v, col3)


def _fin_body(p_ref, o_ref):
    p = p_ref[0] + p_ref[1]
    eps = 1e-6
    v1 = p[:, 0:3]
    v2 = p[:, 8:11]
    v1n = jnp.sqrt(jnp.sum(v1 * v1, axis=1, keepdims=True))
    one = jnp.ones_like(v1n)
    zero = jnp.zeros_like(v1n)
    default = jnp.concatenate([one, zero, zero], axis=1)
    n1 = jnp.where(v1n > eps, v1 / (v1n + eps), default)
    n2p = v2 - jnp.sum(n1 * v2, axis=1, keepdims=True) * n1
    n2n = jnp.sqrt(jnp.sum(n2p * n2p, axis=1, keepdims=True))
    fb = jnp.concatenate([-n1[:, 1:2], n1[:, 0:1], zero], axis=1)
    fb = fb - jnp.sum(n1 * fb, axis=1, keepdims=True) * n1
    fbn = jnp.sqrt(jnp.sum(fb * fb, axis=1, keepdims=True))
    fb = fb / (fbn + eps)
    n2 = jnp.where(n2n > eps, n2p / (n2n + eps), fb)
    c0 = n1[:, 1:2] * n2[:, 2:3] - n1[:, 2:3] * n2[:, 1:2]
    c1 = n1[:, 2:3] * n2[:, 0:1] - n1[:, 0:1] * n2[:, 2:3]
    c2 = n1[:, 0:1] * n2[:, 1:2] - n1[:, 1:2] * n2[:, 0:1]
    n3 = jnp.concatenate([c0, c1, c2], axis=1)
    n3n = jnp.sqrt(jnp.sum(n3 * n3, axis=1, keepdims=True))
    n3 = n3 / (n3n + eps)
    pad7 = jnp.zeros((p.shape[0], 7), jnp.float32)
    o_ref[...] = jnp.concatenate(
        [n1[:, 0:1], n2[:, 0:1], n3[:, 0:1],
         n1[:, 1:2], n2[:, 1:2], n3[:, 1:2],
         n1[:, 2:3], n2[:, 2:3], n3[:, 2:3], pad7], axis=1)


def _finalize(p):
    blk = 1024
    return pl.pallas_call(
        _fin_body,
        grid=(NPAD // blk,),
        in_specs=[pl.BlockSpec((NC, blk, 16), lambda i: (0, i, 0))],
        out_specs=pl.BlockSpec((blk, 16), lambda i: (i, 0)),
        out_shape=jax.ShapeDtypeStruct((NPAD, 16), jnp.float32),
    )(p)


def kernel(x, pos, edge_index, W1a, b1a, W1b, b1b, W2a, b2a, W2b, b2b):
    Wr = jnp.concatenate([W1a[:F], W2a[:F]], axis=1)
    Wc = jnp.concatenate([W1a[F:2 * F], W2a[F:2 * F]], axis=1)
    ones16 = jnp.ones((16, 16), jnp.float32)
    wd128 = jnp.concatenate([W1a[2 * F], W2a[2 * F]])
    WD = jnp.tile(wd128[None, :] / 16.0, (16, 1))
    WM = jnp.zeros((128, 16), jnp.float32)
    WM = WM.at[0:H, 0:8].set(jnp.tile(W1b, (1, 8)))
    WM = WM.at[H:2 * H, 8:16].set(jnp.tile(W2b, (1, 8)))
    BA = jnp.zeros((8, 128), jnp.float32)
    BA = BA.at[0, 0:H].set(b1a)
    BA = BA.at[0, H:2 * H].set(b2a)
    BB = jnp.zeros((8, 16), jnp.float32)
    BB = BB.at[0, 0:8].set(b1b[0])
    BB = BB.at[0, 8:16].set(b2b[0])
    padlen = EPAD - E
    row = jnp.concatenate([edge_index[0], jnp.zeros((padlen,), jnp.int32)])
    col = jnp.concatenate([edge_index[1], jnp.zeros((padlen,), jnp.int32)])
    rowg = row.reshape(NS, NCHT, CH)
    colg = col.reshape(NS, NCHT, CH)
    col3 = col.reshape(NW, NCH, CH)
    p16 = jnp.pad(pos, ((0, 0), (0, 13)))
    tr, tc_ = _node_tables(x, Wr, Wc)
    gr, gc, q = _sc_gather(tr, tc_, p16, rowg, colg)
    v = _edge_math(gr, gc, q, ones16, WD, WM, BA, BB)
    p = _sc_scatter(v, col3)
    o = _finalize(p)
    return o[:N, :9].reshape(N, 3, 3)


# speedup vs baseline: 4.3619x; 1.0723x over previous
import functools

import jax
import jax.numpy as jnp
from jax import lax
from jax.experimental import pallas as pl
from jax.experimental.pallas import tpu as pltpu
from jax.experimental.pallas import tpu_sc as plsc

N = 10000
E = 320000
F = 128
H = 64
NC, NS = 2, 16
NW = NC * NS
CH = 128
EPAD = 327680
PERW = EPAD // NW
NCH = PERW // CH
NPAD = 10240
TW = 144
BE = 4096


def _tables_body(x_ref, wr_ref, wc_ref, tr_ref, tc_ref):
    x = x_ref[...]
    tr_ref[...] = jnp.dot(x, wr_ref[...], preferred_element_type=jnp.float32)
    tc_ref[...] = jnp.dot(x, wc_ref[...], preferred_element_type=jnp.float32)


def _node_tables(x, Wr, Wc):
    blk = 1000
    return pl.pallas_call(
        _tables_body,
        grid=(N // blk,),
        in_specs=[pl.BlockSpec((blk, F), lambda i: (i, 0)),
                  pl.BlockSpec((F, F), lambda i: (0, 0)),
                  pl.BlockSpec((F, F), lambda i: (0, 0))],
        out_specs=[pl.BlockSpec((blk, F), lambda i: (i, 0)),
                   pl.BlockSpec((blk, F), lambda i: (i, 0))],
        out_shape=[jax.ShapeDtypeStruct((N, F), jnp.float32)] * 2,
    )(x, Wr, Wc)


_SC_PARAMS = pltpu.CompilerParams(use_tc_tiling_on_sc=False)


NCHT = 2 * NCH
NCH0 = 120
NCH1 = NCHT - NCH0


def _sc_gather(tr, tc_, p16, row3, col3):
    mesh = plsc.VectorSubcoreMesh(core_axis_name="c", subcore_axis_name="s")

    @functools.partial(
        pl.kernel, mesh=mesh,
        compiler_params=_SC_PARAMS,
        out_type=[jax.ShapeDtypeStruct((EPAD, F), jnp.float32),
                  jax.ShapeDtypeStruct((EPAD, F), jnp.float32),
                  jax.ShapeDtypeStruct((EPAD, 128), jnp.float32)],
        scratch_types=[pltpu.VMEM((NCH0, CH), jnp.int32),
                       pltpu.VMEM((NCH0, CH), jnp.int32),
                       pltpu.VMEM((CH, F), jnp.float32),
                       pltpu.VMEM((CH, F), jnp.float32),
                       pltpu.VMEM((CH, 16), jnp.float32),
                       pltpu.VMEM((CH, 16), jnp.float32),
                       pltpu.VMEM((CH, F), jnp.float32),
                       pltpu.VMEM((CH, F), jnp.float32),
                       pltpu.VMEM((CH, 16), jnp.float32),
                       pltpu.VMEM((CH, 16), jnp.float32),
                       pltpu.SemaphoreType.DMA,
                       pltpu.SemaphoreType.DMA,
                       pltpu.SemaphoreType.DMA,
                       pltpu.SemaphoreType.DMA,
                       pltpu.SemaphoreType.DMA,
                       pltpu.SemaphoreType.DMA,
                       pltpu.SemaphoreType.DMA,
                       pltpu.SemaphoreType.DMA,
                       pltpu.SemaphoreType.DMA,
                       pltpu.SemaphoreType.DMA,
                       pltpu.SemaphoreType.DMA,
                       pltpu.SemaphoreType.DMA,
                       pltpu.SemaphoreType.DMA],
    )
    def k(tr_hbm, tc_hbm, p16_hbm, row_hbm, col_hbm, gr_hbm, gc_hbm, q_hbm,
          idxr, idxc, br0, bc0, pr0, pc0, br1, bc1, pr1, pc1,
          gsr0, gsc0, gpr0, gpc0, gsr1, gsc1, gpr1, gpc1,
          wsr0, wsq0, wsr1, wsq1, isem):
        cid = lax.axis_index("c")
        sid = lax.axis_index("s")
        pair_base = sid * (NCHT * CH)
        bufs = ((br0, bc0, pr0, pc0, gsr0, gsc0, gpr0, gpc0, wsr0, wsq0),
                (br1, bc1, pr1, pc1, gsr1, gsc1, gpr1, gpc1, wsr1, wsq1))

        def pipeline(nch, chunk0, ebase):
            pltpu.async_copy(row_hbm.at[sid, pl.ds(chunk0, nch)],
                             idxr.at[pl.ds(0, nch)], isem).wait()
            pltpu.async_copy(col_hbm.at[sid, pl.ds(chunk0, nch)],
                             idxc.at[pl.ds(0, nch)], isem).wait()

            def _gather_copies(j, p):
                br, bc, pr, pc, gsr, gsc, gpr, gpc = bufs[p][:8]
                return (
                    pltpu.make_async_copy(tr_hbm.at[idxr.at[j]], br, gsr),
                    pltpu.make_async_copy(tc_hbm.at[idxc.at[j]], bc, gsc),
                    pltpu.make_async_copy(p16_hbm.at[idxr.at[j]], pr, gpr),
                    pltpu.make_async_copy(p16_hbm.at[idxc.at[j]], pc, gpc),
                )

            def start_gather(j, p):
                for c in _gather_copies(j, p):
                    c.start()

            def wait_gather(j, p):
                for c in _gather_copies(j, p):
                    c.wait()

            def _write_copies(j, p):
                br, bc, pr, pc = bufs[p][:4]
                wsr, wsq = bufs[p][8:]
                off = ebase + j * CH
                rows = pl.ds(off, CH)
                return (
                    pltpu.make_async_copy(br, gr_hbm.at[rows], wsr),
                    pltpu.make_async_copy(bc, gc_hbm.at[rows], wsr),
                    pltpu.make_async_copy(
                        pr, q_hbm.at[rows, pl.ds(0, 16)], wsq),
                    pltpu.make_async_copy(
                        pc, q_hbm.at[rows, pl.ds(16, 16)], wsq),
                )

            def start_write(j, p):
                for c in _write_copies(j, p):
                    c.start()

            def wait_write(j, p):
                for c in _write_copies(j, p):
                    c.wait()

            start_gather(0, 0)
            start_gather(1, 1)
            wait_gather(0, 0)
            start_write(0, 0)
            wait_gather(1, 1)
            start_write(1, 1)

            @pl.loop(1, nch // 2)
            def _(i):
                j0 = 2 * i
                for p in range(2):
                    j = j0 + p
                    wait_write(j, p)
                    start_gather(j, p)
                    wait_gather(j, p)
                    start_write(j, p)

            wait_write(nch - 2, 0)
            wait_write(nch - 1, 1)

        @pl.when(cid == 0)
        def _():
            pipeline(NCH0, 0, pair_base)

        @pl.when(cid == 1)
        def _():
            pipeline(NCH1, NCH0, pair_base + NCH0 * CH)

    return k(tr, tc_, p16, row3, col3)


def _edge_body(gr_ref, gc_ref, q_ref, on_ref, wd_ref, wm_ref, ba_ref, bb_ref,
               v_ref):
    q = q_ref[...]
    d3 = q[:, 0:16] - q[:, 16:32]
    dist2 = jnp.dot(d3 * d3, on_ref[...],
                    preferred_element_type=jnp.float32)
    dist16 = jnp.sqrt(dist2)
    distwd = jnp.dot(dist16, wd_ref[...],
                     preferred_element_type=jnp.float32)
    h = gr_ref[...] + gc_ref[...] + distwd + ba_ref[0:1, :]
    s = h * jax.nn.sigmoid(h)
    mes = jnp.dot(s, wm_ref[...],
                  preferred_element_type=jnp.float32) + bb_ref[0:1, :]
    rmax = 4.5
    t = jnp.clip(dist16, 0.0, rmax) / rmax
    t2 = t * t
    t4 = t2 * t2
    t5 = t4 * t
    t6 = t5 * t
    t7 = t6 * t
    coe = 1.0 - 21.0 * t5 + 35.0 * t6 - 15.0 * t7
    fac = (coe / (dist16 + 1e-6)) * mes
    d3pair = jnp.concatenate([d3[:, 0:8], d3[:, 0:8]], axis=1)
    v_ref[...] = d3pair * fac


def _edge_math(gr, gc, q, ones16, WD, WM, BA, BB):
    return pl.pallas_call(
        _edge_body,
        grid=(EPAD // BE,),
        in_specs=[pl.BlockSpec((BE, F), lambda i: (i, 0)),
                  pl.BlockSpec((BE, F), lambda i: (i, 0)),
                  pl.BlockSpec((BE, 128), lambda i: (i, 0)),
                  pl.BlockSpec((16, 16), lambda i: (0, 0)),
                  pl.BlockSpec((16, 128), lambda i: (0, 0)),
                  pl.BlockSpec((128, 16), lambda i: (0, 0)),
                  pl.BlockSpec((8, 128), lambda i: (0, 0)),
                  pl.BlockSpec((8, 16), lambda i: (0, 0))],
        out_specs=pl.BlockSpec((BE, 16), lambda i: (i, 0)),
        out_shape=jax.ShapeDtypeStruct((EPAD, 16), jnp.float32),
    )(gr, gc, q, ones16, WD, WM, BA, BB)


def _sc_scatter(v, col3):
    mesh = plsc.VectorSubcoreMesh(core_axis_name="c", subcore_axis_name="s")
    ZR = NPAD // NS

    @functools.partial(
        pl.kernel, mesh=mesh,
        compiler_params=_SC_PARAMS,
        out_type=jax.ShapeDtypeStruct((NC, NPAD, 16), jnp.float32),
        scratch_types=[pltpu.VMEM((CH, 16), jnp.float32),
                       pltpu.VMEM((CH, 16), jnp.float32),
                       pltpu.VMEM((NCH, CH), jnp.int32),
                       pltpu.VMEM_SHARED((NPAD, 16), jnp.float32),
                       pltpu.SemaphoreType.DMA,
                       pltpu.SemaphoreType.DMA,
                       pltpu.SemaphoreType.DMA,
                       pltpu.SemaphoreType.DMA,
                       pltpu.SemaphoreType.DMA],
    )
    def k(v_hbm, col_hbm, out_hbm, vb0, vb1, idxbuf, acc,
          ls0, ls1, ss0, ss1, isem):
        cid = lax.axis_index("c")
        sid = lax.axis_index("s")
        wid = sid * NC + cid
        base = wid * PERW

        @pl.loop(0, CH)
        def _(i):
            vb0[i, :] = jnp.zeros((16,), jnp.float32)

        @pl.loop(0, ZR // CH)
        def _(j):
            pltpu.sync_copy(vb0, acc.at[pl.ds(sid * ZR + j * CH, CH)])

        pltpu.async_copy(col_hbm.at[wid], idxbuf, isem).wait()
        plsc.subcore_barrier()

        bufs = ((vb0, ls0, ss0), (vb1, ls1, ss1))

        def start_load(j, p):
            vb, ls, _ = bufs[p]
            off = base + j * CH
            pltpu.make_async_copy(v_hbm.at[pl.ds(off, CH)], vb, ls).start()

        def wait_load(j, p):
            vb, ls, _ = bufs[p]
            off = base + j * CH
            pltpu.make_async_copy(v_hbm.at[pl.ds(off, CH)], vb, ls).wait()

        def start_scatter(j, p):
            vb, _, ss = bufs[p]
            pltpu.async_copy(vb, acc.at[idxbuf.at[j]], ss, add=True)

        def wait_scatter(j, p):
            vb, _, ss = bufs[p]
            pltpu.make_async_copy(vb, acc.at[idxbuf.at[j]], ss).wait()

        start_load(0, 0)
        start_load(1, 1)

        @pl.loop(0, NCH // 2)
        def _(i):
            j0 = 2 * i
            for p in range(2):
                j = j0 + p
                wait_load(j, p)
                start_scatter(j, p)
                wait_scatter(j, p)

                @pl.when(j + 2 < NCH)
                def _():
                    start_load(j + 2, p)

        plsc.subcore_barrier()
        pltpu.sync_copy(acc.at[pl.ds(sid * ZR, ZR)],
                        out_hbm.at[cid, pl.ds(sid * ZR, ZR)])

    return k(v, col3)


def _fin_body(p_ref, o_ref):
    p = p_ref[0] + p_ref[1]
    eps = 1e-6
    v1 = p[:, 0:3]
    v2 = p[:, 8:11]
    v1n = jnp.sqrt(jnp.sum(v1 * v1, axis=1, keepdims=True))
    one = jnp.ones_like(v1n)
    zero = jnp.zeros_like(v1n)
    default = jnp.concatenate([one, zero, zero], axis=1)
    n1 = jnp.where(v1n > eps, v1 / (v1n + eps), default)
    n2p = v2 - jnp.sum(n1 * v2, axis=1, keepdims=True) * n1
    n2n = jnp.sqrt(jnp.sum(n2p * n2p, axis=1, keepdims=True))
    fb = jnp.concatenate([-n1[:, 1:2], n1[:, 0:1], zero], axis=1)
    fb = fb - jnp.sum(n1 * fb, axis=1, keepdims=True) * n1
    fbn = jnp.sqrt(jnp.sum(fb * fb, axis=1, keepdims=True))
    fb = fb / (fbn + eps)
    n2 = jnp.where(n2n > eps, n2p / (n2n + eps), fb)
    c0 = n1[:, 1:2] * n2[:, 2:3] - n1[:, 2:3] * n2[:, 1:2]
    c1 = n1[:, 2:3] * n2[:, 0:1] - n1[:, 0:1] * n2[:, 2:3]
    c2 = n1[:, 0:1] * n2[:, 1:2] - n1[:, 1:2] * n2[:, 0:1]
    n3 = jnp.concatenate([c0, c1, c2], axis=1)
    n3n = jnp.sqrt(jnp.sum(n3 * n3, axis=1, keepdims=True))
    n3 = n3 / (n3n + eps)
    pad7 = jnp.zeros((p.shape[0], 7), jnp.float32)
    o_ref[...] = jnp.concatenate(
        [n1[:, 0:1], n2[:, 0:1], n3[:, 0:1],
         n1[:, 1:2], n2[:, 1:2], n3[:, 1:2],
         n1[:, 2:3], n2[:, 2:3], n3[:, 2:3], pad7], axis=1)


def _finalize(p):
    blk = 1024
    return pl.pallas_call(
        _fin_body,
        grid=(NPAD // blk,),
        in_specs=[pl.BlockSpec((NC, blk, 16), lambda i: (0, i, 0))],
        out_specs=pl.BlockSpec((blk, 16), lambda i: (i, 0)),
        out_shape=jax.ShapeDtypeStruct((NPAD, 16), jnp.float32),
    )(p)


def kernel(x, pos, edge_index, W1a, b1a, W1b, b1b, W2a, b2a, W2b, b2b):
    Wr = jnp.concatenate([W1a[:F], W2a[:F]], axis=1)
    Wc = jnp.concatenate([W1a[F:2 * F], W2a[F:2 * F]], axis=1)
    ones16 = jnp.ones((16, 16), jnp.float32)
    wd128 = jnp.concatenate([W1a[2 * F], W2a[2 * F]])
    WD = jnp.tile(wd128[None, :] / 16.0, (16, 1))
    WM = jnp.zeros((128, 16), jnp.float32)
    WM = WM.at[0:H, 0:8].set(jnp.tile(W1b, (1, 8)))
    WM = WM.at[H:2 * H, 8:16].set(jnp.tile(W2b, (1, 8)))
    BA = jnp.zeros((8, 128), jnp.float32)
    BA = BA.at[0, 0:H].set(b1a)
    BA = BA.at[0, H:2 * H].set(b2a)
    BB = jnp.zeros((8, 16), jnp.float32)
    BB = BB.at[0, 0:8].set(b1b[0])
    BB = BB.at[0, 8:16].set(b2b[0])
    padlen = EPAD - E
    row = jnp.concatenate([edge_index[0], jnp.zeros((padlen,), jnp.int32)])
    col = jnp.concatenate([edge_index[1], jnp.zeros((padlen,), jnp.int32)])
    rowg = row.reshape(NS, NCHT, CH)
    colg = col.reshape(NS, NCHT, CH)
    col3 = col.reshape(NW, NCH, CH)
    p16 = jnp.pad(pos, ((0, 0), (0, 13)))
    tr, tc_ = _node_tables(x, Wr, Wc)
    gr, gc, q = _sc_gather(tr, tc_, p16, rowg, colg)
    v = _edge_math(gr, gc, q, ones16, WD, WM, BA, BB)
    p = _sc_scatter(v, col3)
    o = _finalize(p)
    return o[:N, :9].reshape(N, 3, 3)
